# Initial kernel scaffold; baseline (speedup 1.0000x reference)
import jax
import jax.numpy as jnp
from jax.experimental import pallas as pl


def kernel(user_feat, item_feat, Wu, Wi, Wg_u, Wg_i, i_te, i_te_k, u_te, u_te_k, item_neighbors, item_nbr_time, user_neighbors, user_nbr_time):
    raise NotImplementedError("write your pallas kernel here")


# speedup vs baseline: 1.7079x; 1.7079x over previous
import functools

import jax
import jax.numpy as jnp
from jax import lax
from jax.experimental import pallas as pl

D = 128
DEG = 32
N = 10000
NPAD = 10240
BLK = 128
PBLK = 256


def _proj_body(feat_ref, w_ref, te_ref, h_ref, tp_ref):
    h = jnp.dot(feat_ref[...], w_ref[...].T, preferred_element_type=jnp.float32)
    h_ref[...] = h
    tp_ref[...] = jnp.dot(h, te_ref[...].T, preferred_element_type=jnp.float32)


def _proj(feat, w, te32):
    n = feat.shape[0]
    grid = n // PBLK
    return pl.pallas_call(
        _proj_body,
        grid=(grid,),
        in_specs=[
            pl.BlockSpec((PBLK, D), lambda i: (i, 0)),
            pl.BlockSpec((D, D), lambda i: (0, 0)),
            pl.BlockSpec((DEG, D), lambda i: (0, 0)),
        ],
        out_specs=[
            pl.BlockSpec((PBLK, D), lambda i: (i, 0)),
            pl.BlockSpec((PBLK, DEG), lambda i: (i, 0)),
        ],
        out_shape=[
            jax.ShapeDtypeStruct((n, D), jnp.float32),
            jax.ShapeDtypeStruct((n, DEG), jnp.float32),
        ],
    )(feat, w, te32)


def _att_body(mail_ref, t_ref, tp_ref, dsth_ref, feat_ref, tek_ref, wg_ref,
              out_ref):
    mail = mail_ref[...]
    t = t_ref[...]
    tp = tp_ref[...]
    dsth = dsth_ref[...]
    scale = jnp.sqrt(jnp.float32(D))

    tj = t[:, :, None]
    tk = t[:, None, :]
    kk = lax.broadcasted_iota(jnp.int32, (1, DEG, DEG), 2)
    jj = lax.broadcasted_iota(jnp.int32, (1, DEG, DEG), 1)
    gt = (tk > tj) | ((tk == tj) & (kk > jj))
    re_order = jnp.sum(gt.astype(jnp.int32), axis=2)
    ge_first = (tk > tj) | ((tk == tj) & (kk < jj))
    fm = (jnp.sum(ge_first.astype(jnp.int32), axis=2) == 0)

    onehot = (re_order[:, :, None] ==
              lax.broadcasted_iota(jnp.int32, (1, 1, DEG), 2))
    onehot_f = onehot.astype(jnp.float32)

    e_te = jnp.sum(onehot_f * tp[:, None, :], axis=2)
    e_mail = jnp.sum(mail * dsth[:, None, :], axis=2)
    e = (e_te + e_mail) / scale
    alpha = jax.nn.softmax(e, axis=1)
    h_long_mail = jnp.sum(alpha[:, :, None] * mail, axis=1)
    beta = jnp.sum(onehot_f * alpha[:, :, None], axis=1)
    h_long = h_long_mail + jnp.dot(beta, tek_ref[...],
                                   preferred_element_type=jnp.float32)

    last_em = jnp.sum(jnp.where(fm[:, :, None], mail, 0.0), axis=1)
    e1 = jnp.sum(mail * last_em[:, None, :], axis=2) / scale
    alpha1 = jax.nn.softmax(e1, axis=1)
    h_short = jnp.sum(alpha1[:, :, None] * mail, axis=1)

    msg = jnp.concatenate([h_long, h_short], axis=1)
    new = lax.dot_general(msg, wg_ref[...],
                          (((1,), (1,)), ((), ())),
                          preferred_element_type=jnp.float32)
    x = new + feat_ref[...]
    out_ref[...] = jnp.where(x > 0, x, jnp.exp(x) - 1.0)


def _attention(mail, t, tp, dsth, feat, tek32, wg):
    n = mail.shape[0]
    grid = n // BLK
    return pl.pallas_call(
        _att_body,
        grid=(grid,),
        in_specs=[
            pl.BlockSpec((BLK, DEG, D), lambda i: (i, 0, 0)),
            pl.BlockSpec((BLK, DEG), lambda i: (i, 0)),
            pl.BlockSpec((BLK, DEG), lambda i: (i, 0)),
            pl.BlockSpec((BLK, D), lambda i: (i, 0)),
            pl.BlockSpec((BLK, D), lambda i: (i, 0)),
            pl.BlockSpec((DEG, D), lambda i: (0, 0)),
            pl.BlockSpec((D, 2 * D), lambda i: (0, 0)),
        ],
        out_specs=pl.BlockSpec((BLK, D), lambda i: (i, 0)),
        out_shape=jax.ShapeDtypeStruct((n, D), jnp.float32),
    )(mail, t, tp, dsth, feat, tek32, wg)


def _pad_rows(x, npad):
    return jnp.pad(x, ((0, npad - x.shape[0]),) + ((0, 0),) * (x.ndim - 1))


def kernel(user_feat, item_feat, Wu, Wi, Wg_u, Wg_i, i_te, i_te_k, u_te,
           u_te_k, item_neighbors, item_nbr_time, user_neighbors,
           user_nbr_time):
    user_h, tp_u = _proj(_pad_rows(user_feat, NPAD), Wu, u_te[:DEG])
    item_h, tp_i = _proj(_pad_rows(item_feat, NPAD), Wi, i_te[:DEG])

    def side(src_h, nbrs, times, tp, dst_h, dst_feat, tek, wg):
        mail = jnp.take(src_h, nbrs, axis=0)
        mail = _pad_rows(mail, NPAD)
        t = _pad_rows(times.astype(jnp.int32), NPAD)
        out = _attention(mail, t, tp, dst_h,
                         _pad_rows(dst_feat, NPAD), tek[:DEG], wg)
        return out[:N]

    item_out = side(user_h, item_neighbors, item_nbr_time, tp_i, item_h,
                    item_feat, i_te_k, Wg_i)
    user_out = side(item_h, user_neighbors, user_nbr_time, tp_u, user_h,
                    user_feat, u_te_k, Wg_u)
    return user_out, item_out
